# baseline (speedup 1.0000x reference)
import functools

import jax
import jax.numpy as jnp
from jax import lax
from jax.experimental import pallas as pl
from jax.experimental.pallas import tpu as pltpu

_NUM_DIFF_LAYERS = 10
_C = 0.15
_BM = 400

_A_SCALE = 1024.0
_F8 = jnp.float8_e4m3fn
_DN = (((1,), (0,)), ((), ()))


def _quantize_kernel(a_ref, out_ref):
    out_ref[...] = ((a_ref[...] * _A_SCALE).astype(_F8))[None]


def _quantize(a, bm):
    n = a.shape[0]
    return pl.pallas_call(
        _quantize_kernel,
        grid=(n // bm,),
        in_specs=[pl.BlockSpec((bm, n), lambda i: (i, 0))],
        out_specs=pl.BlockSpec((1, bm, n), lambda i: (i, 0, 0)),
        out_shape=jax.ShapeDtypeStruct((n // bm, bm, n), _F8),
    )(a)


def _diffusion_kernel(ap_ref, am_ref, pm0_ref, tx_ref, p_ref, m_ref,
                      s0_ref, s1_ref, *, d, bm, nsteps):
    s = pl.program_id(0)
    i = pl.program_id(1)

    @pl.when(jnp.logical_and(s == 0, i == 0))
    def _():
        s0_ref[...] = pm0_ref[...]

    def body(cur_ref, nxt_ref):
        ap = ap_ref[0].astype(jnp.bfloat16)
        am = am_ref[0].astype(jnp.bfloat16)
        pm = cur_ref[...]
        y1 = lax.dot_general(ap, pm, _DN,
                             preferred_element_type=jnp.float32)
        y2 = lax.dot_general(am, pm, _DN,
                             preferred_element_type=jnp.float32)
        inv = 1.0 / _A_SCALE
        tx = tx_ref[pl.ds(i * bm, bm), :]
        newp = (y1[:, :d] + y2[:, d:]) * inv + tx
        newm = (y2[:, :d] + y1[:, d:]) * inv
        p_ref[...] = newp
        m_ref[...] = newm
        nxt_ref[pl.ds(i * bm, bm), :] = jnp.concatenate(
            [newp, newm], axis=1).astype(jnp.bfloat16)

    @pl.when(lax.rem(s, 2) == 0)
    def _():
        body(s0_ref, s1_ref)

    @pl.when(lax.rem(s, 2) == 1)
    def _():
        body(s1_ref, s0_ref)


def _diffusion(ap8, am8, pm0, tx, bm, nsteps):
    n = pm0.shape[0]
    d = tx.shape[1]

    def out_idx(s, i):
        return (jnp.where(s == nsteps - 1, i, 0), 0)

    return pl.pallas_call(
        functools.partial(_diffusion_kernel, d=d, bm=bm, nsteps=nsteps),
        grid=(nsteps, n // bm),
        in_specs=[
            pl.BlockSpec((1, bm, n), lambda s, i: (i, 0, 0)),
            pl.BlockSpec((1, bm, n), lambda s, i: (i, 0, 0)),
            pl.BlockSpec((n, 2 * d), lambda s, i: (0, 0)),
            pl.BlockSpec((n, d), lambda s, i: (0, 0)),
        ],
        out_specs=[
            pl.BlockSpec((bm, d), out_idx),
            pl.BlockSpec((bm, d), out_idx),
        ],
        out_shape=[
            jax.ShapeDtypeStruct((n, d), jnp.float32),
            jax.ShapeDtypeStruct((n, d), jnp.float32),
        ],
        scratch_shapes=[
            pltpu.VMEM((n, 2 * d), jnp.bfloat16),
            pltpu.VMEM((n, 2 * d), jnp.bfloat16),
        ],
    )(ap8, am8, pm0, tx)


def kernel(nApT, nAmT, X):
    m0 = jax.random.uniform(jax.random.key(1), X.shape, dtype=jnp.float32,
                            minval=-1.0, maxval=1.0)
    tx = _C * X
    pm0 = jnp.concatenate([X, m0], axis=1).astype(jnp.bfloat16)
    ap8 = _quantize(nApT, _BM)
    am8 = _quantize(nAmT, _BM)
    return _diffusion(ap8, am8, pm0, tx, _BM, _NUM_DIFF_LAYERS)
